# baseline (speedup 1.0000x reference)
import jax
import jax.numpy as jnp
from jax.experimental import pallas as pl


SEQ_BLK = 512


def _add_kernel(x_ref, w_ref, o_ref):
    o_ref[...] = x_ref[...] + w_ref[...][None, :, :]


def kernel(inputs, W):
    batch, seq_len, dim = inputs.shape
    grid = (seq_len // SEQ_BLK, batch)
    return pl.pallas_call(
        _add_kernel,
        grid=grid,
        in_specs=[
            pl.BlockSpec((1, SEQ_BLK, dim), lambda i, b: (b, i, 0)),
            pl.BlockSpec((SEQ_BLK, dim), lambda i, b: (i, 0)),
        ],
        out_specs=pl.BlockSpec((1, SEQ_BLK, dim), lambda i, b: (b, i, 0)),
        out_shape=jax.ShapeDtypeStruct((batch, seq_len, dim), inputs.dtype),
    )(inputs, W)


# speedup vs baseline: 1.1483x; 1.1483x over previous
import jax
import jax.numpy as jnp
from jax.experimental import pallas as pl


SEQ_BLK = 512


def _add_kernel(x_ref, w_ref, o_ref):
    o_ref[...] = x_ref[...] + w_ref[...][None, :, :]


def kernel(inputs, W):
    batch, seq_len, dim = inputs.shape
    grid = (seq_len // SEQ_BLK,)
    return pl.pallas_call(
        _add_kernel,
        grid=grid,
        in_specs=[
            pl.BlockSpec((batch, SEQ_BLK, dim), lambda i: (0, i, 0)),
            pl.BlockSpec((SEQ_BLK, dim), lambda i: (i, 0)),
        ],
        out_specs=pl.BlockSpec((batch, SEQ_BLK, dim), lambda i: (0, i, 0)),
        out_shape=jax.ShapeDtypeStruct((batch, seq_len, dim), inputs.dtype),
    )(inputs, W)
